# Initial kernel scaffold; baseline (speedup 1.0000x reference)
import jax
import jax.numpy as jnp
from jax.experimental import pallas as pl


def kernel(z, emb):
    raise NotImplementedError("write your pallas kernel here")


# speedup vs baseline: 1.0918x; 1.0918x over previous
import functools

import jax
import jax.numpy as jnp
from jax import lax
from jax.experimental import pallas as pl
from jax.experimental.pallas import tpu as pltpu
from jax.experimental.pallas import tpu_sc as plsc

_BETA = 0.1


_BM = 256


_KT = 2736


def _round_bf16(x):
    u = lax.bitcast_convert_type(x, jnp.uint32)
    r = u + jnp.uint32(0x7FFF) + ((u >> 16) & jnp.uint32(1))
    return lax.bitcast_convert_type(r & jnp.uint32(0xFFFF0000), jnp.float32)


def _argmin_body(z_ref, emb_ref, z2_ref, e2a_ref, e2b_ref, e2c_ref, idx_ref):
    k = emb_ref.shape[0]
    zb = z_ref[...].astype(jnp.bfloat16)
    z2 = z2_ref[...]
    acc_v = jnp.full((z_ref.shape[0], 1), jnp.inf, dtype=jnp.float32)
    acc_i = jnp.zeros((z_ref.shape[0], 1), dtype=jnp.int32)
    for t, e2_ref in enumerate((e2a_ref, e2b_ref, e2c_ref)):
        lo = t * _KT
        w = min(_KT, k - lo)
        eb = emb_ref[pl.ds(lo, w), :].astype(jnp.bfloat16)
        mm = lax.dot_general(
            zb, eb,
            dimension_numbers=(((1,), (1,)), ((), ())),
            preferred_element_type=jnp.float32,
        )
        dist = 2.0 * mm
        dist = dist + z2
        dist = dist + e2_ref[...]
        m = jnp.min(dist, axis=1, keepdims=True)
        iota = lax.broadcasted_iota(jnp.int32, dist.shape, 1) + lo
        cand = jnp.where(dist == m, iota, k)
        i = jnp.min(cand, axis=1, keepdims=True)
        keep = (acc_v < m) | ((acc_v == m) & (acc_i < i))
        acc_v = _round_bf16(jnp.where(keep, acc_v, m))
        acc_i = jnp.where(keep, acc_i, i)
    idx_ref[...] = acc_i


def _argmin_call(zf, emb, z2, e2):
    m, c = zf.shape
    k = emb.shape[0]
    grid = (m // _BM,)
    return pl.pallas_call(
        _argmin_body,
        grid=grid,
        in_specs=[
            pl.BlockSpec((_BM, c), lambda i: (i, 0)),
            pl.BlockSpec((k, c), lambda i: (0, 0)),
            pl.BlockSpec((_BM, 1), lambda i: (i, 0)),
            pl.BlockSpec((1, _KT), lambda i: (0, 0)),
            pl.BlockSpec((1, _KT), lambda i: (0, 0)),
            pl.BlockSpec((1, k - 2 * _KT), lambda i: (0, 0)),
        ],
        out_specs=pl.BlockSpec((_BM, 1), lambda i: (i, 0)),
        out_shape=jax.ShapeDtypeStruct((m, 1), jnp.int32),
    )(zf, emb, z2, e2[:, :_KT], e2[:, _KT:2 * _KT], e2[:, 2 * _KT:])


_NC, _NS, _L = 2, 16, 16
_NW = _NC * _NS


def _sc_body(z_hbm, emb_hbm, idx_hbm, out_hbm, part_hbm,
             idx_v, rows_v, z_v, out_v, acc_v, gsem):
    m, c = z_hbm.shape
    rows_per_w = m // _NW
    ch = 128
    n_chunks = rows_per_w // ch
    groups = c // _L

    wid = lax.axis_index("s") * _NC + lax.axis_index("c")
    base = wid * rows_per_w
    pltpu.sync_copy(idx_hbm.at[pl.ds(base, rows_per_w)], idx_v)

    def do_chunk(ci, acc):
        pltpu.async_copy(
            emb_hbm.at[idx_v.at[pl.ds(ci * ch, ch)]], rows_v, gsem
        ).wait()
        pltpu.sync_copy(z_hbm.at[pl.ds(base + ci * ch, ch)], z_v)

        def do_row(r, acc):
            for g in range(groups):
                q = rows_v[r, pl.ds(g * _L, _L)]
                zz = z_v[r, pl.ds(g * _L, _L)]
                d = q - zz
                out_v[r, pl.ds(g * _L, _L)] = zz + d
                acc = acc + d * d
            return acc

        acc = lax.fori_loop(0, ch, do_row, acc)
        pltpu.sync_copy(out_v, out_hbm.at[pl.ds(base + ci * ch, ch)])
        return acc

    acc = lax.fori_loop(0, n_chunks, do_chunk, jnp.zeros((_L,), jnp.float32))
    acc_v[...] = acc
    pltpu.sync_copy(acc_v, part_hbm.at[wid])


def _sc_call(zf, emb, idx):
    m, c = zf.shape
    rows_per_w = m // _NW
    ch = 128
    mesh = plsc.VectorSubcoreMesh(core_axis_name="c", subcore_axis_name="s")
    fn = pl.kernel(
        _sc_body,
        out_type=[
            jax.ShapeDtypeStruct((m, c), jnp.float32),
            jax.ShapeDtypeStruct((_NW, _L), jnp.float32),
        ],
        mesh=mesh,
        scratch_types=[
            pltpu.VMEM((rows_per_w,), jnp.int32),
            pltpu.VMEM((ch, c), jnp.float32),
            pltpu.VMEM((ch, c), jnp.float32),
            pltpu.VMEM((ch, c), jnp.float32),
            pltpu.VMEM((_L,), jnp.float32),
            pltpu.SemaphoreType.DMA,
        ],
    )
    return fn(zf, emb, idx)


def kernel(z, emb):
    shape = z.shape
    c = shape[-1]
    m = z.size // c
    zf = z.reshape(m, c)
    z2 = jnp.sum(zf ** 2, axis=1, keepdims=True)
    e2 = jnp.sum(emb ** 2, axis=1)[None, :]
    idx = _argmin_call(zf, emb, z2, e2).reshape(m)
    out_f, parts = _sc_call(zf, emb, idx)
    mean_sq = jnp.sum(parts) / (m * c)
    loss = mean_sq + mean_sq * _BETA
    return (out_f.reshape(shape), loss)


# speedup vs baseline: 1.1476x; 1.0510x over previous
import functools

import jax
import jax.numpy as jnp
from jax import lax
from jax.experimental import pallas as pl
from jax.experimental.pallas import tpu as pltpu
from jax.experimental.pallas import tpu_sc as plsc

_BETA = 0.1


_BM = 256


_KT = 2736


def _round_bf16(x):
    u = lax.bitcast_convert_type(x, jnp.uint32)
    r = u + jnp.uint32(0x7FFF) + ((u >> 16) & jnp.uint32(1))
    return lax.bitcast_convert_type(r & jnp.uint32(0xFFFF0000), jnp.float32)


def _argmin_body(z_ref, emb_ref, z2_ref, e2a_ref, e2b_ref, e2c_ref, idx_ref):
    k = emb_ref.shape[0]
    zb = (z_ref[...] * 2.0).astype(jnp.bfloat16)
    z2 = z2_ref[...]
    acc_v = jnp.full((z_ref.shape[0], 1), jnp.inf, dtype=jnp.float32)
    acc_i = jnp.zeros((z_ref.shape[0], 1), dtype=jnp.int32)
    for t, e2_ref in enumerate((e2a_ref, e2b_ref, e2c_ref)):
        lo = t * _KT
        w = min(_KT, k - lo)
        eb = emb_ref[pl.ds(lo, w), :].astype(jnp.bfloat16)
        dist = lax.dot_general(
            zb, eb,
            dimension_numbers=(((1,), (1,)), ((), ())),
            preferred_element_type=jnp.float32,
        )
        dist = dist + z2
        dist = dist + e2_ref[...]
        i = jnp.argmin(dist, axis=1)[:, None].astype(jnp.int32) + lo
        m = jnp.min(dist, axis=1, keepdims=True)
        keep = (acc_v < m) | ((acc_v == m) & (acc_i < i))
        acc_v = _round_bf16(jnp.where(keep, acc_v, m))
        acc_i = jnp.where(keep, acc_i, i)
    idx_ref[...] = acc_i


def _argmin_call(zf, emb, z2, e2):
    m, c = zf.shape
    k = emb.shape[0]
    grid = (m // _BM,)
    return pl.pallas_call(
        _argmin_body,
        grid=grid,
        in_specs=[
            pl.BlockSpec((_BM, c), lambda i: (i, 0)),
            pl.BlockSpec((k, c), lambda i: (0, 0)),
            pl.BlockSpec((_BM, 1), lambda i: (i, 0)),
            pl.BlockSpec((1, _KT), lambda i: (0, 0)),
            pl.BlockSpec((1, _KT), lambda i: (0, 0)),
            pl.BlockSpec((1, k - 2 * _KT), lambda i: (0, 0)),
        ],
        out_specs=pl.BlockSpec((_BM, 1), lambda i: (i, 0)),
        out_shape=jax.ShapeDtypeStruct((m, 1), jnp.int32),
    )(zf, emb, z2, e2[:, :_KT], e2[:, _KT:2 * _KT], e2[:, 2 * _KT:])


_NC, _NS, _L = 2, 16, 16
_NW = _NC * _NS


def _sc_body(z_hbm, emb_hbm, idx_hbm, out_hbm, part_hbm,
             idx_v, rows_v, z_v, out_v, acc_v, gsem):
    m, c = z_hbm.shape
    rows_per_w = m // _NW
    ch = 128
    n_chunks = rows_per_w // ch
    groups = c // _L

    wid = lax.axis_index("s") * _NC + lax.axis_index("c")
    base = wid * rows_per_w
    pltpu.sync_copy(idx_hbm.at[pl.ds(base, rows_per_w)], idx_v)

    def do_chunk(ci, acc):
        pltpu.async_copy(
            emb_hbm.at[idx_v.at[pl.ds(ci * ch, ch)]], rows_v, gsem
        ).wait()
        pltpu.sync_copy(z_hbm.at[pl.ds(base + ci * ch, ch)], z_v)

        def do_row(r, acc):
            for g in range(groups):
                q = rows_v[r, pl.ds(g * _L, _L)]
                zz = z_v[r, pl.ds(g * _L, _L)]
                d = q - zz
                out_v[r, pl.ds(g * _L, _L)] = zz + d
                acc = acc + d * d
            return acc

        acc = lax.fori_loop(0, ch, do_row, acc)
        pltpu.sync_copy(out_v, out_hbm.at[pl.ds(base + ci * ch, ch)])
        return acc

    acc = lax.fori_loop(0, n_chunks, do_chunk, jnp.zeros((_L,), jnp.float32))
    acc_v[...] = acc
    pltpu.sync_copy(acc_v, part_hbm.at[wid])


def _sc_call(zf, emb, idx):
    m, c = zf.shape
    rows_per_w = m // _NW
    ch = 128
    mesh = plsc.VectorSubcoreMesh(core_axis_name="c", subcore_axis_name="s")
    fn = pl.kernel(
        _sc_body,
        out_type=[
            jax.ShapeDtypeStruct((m, c), jnp.float32),
            jax.ShapeDtypeStruct((_NW, _L), jnp.float32),
        ],
        mesh=mesh,
        scratch_types=[
            pltpu.VMEM((rows_per_w,), jnp.int32),
            pltpu.VMEM((ch, c), jnp.float32),
            pltpu.VMEM((ch, c), jnp.float32),
            pltpu.VMEM((ch, c), jnp.float32),
            pltpu.VMEM((_L,), jnp.float32),
            pltpu.SemaphoreType.DMA,
        ],
    )
    return fn(zf, emb, idx)


def kernel(z, emb):
    shape = z.shape
    c = shape[-1]
    m = z.size // c
    zf = z.reshape(m, c)
    z2 = jnp.sum(zf ** 2, axis=1, keepdims=True)
    e2 = jnp.sum(emb ** 2, axis=1)[None, :]
    idx = _argmin_call(zf, emb, z2, e2).reshape(m)
    out_f, parts = _sc_call(zf, emb, idx)
    mean_sq = jnp.sum(parts) / (m * c)
    loss = mean_sq + mean_sq * _BETA
    return (out_f.reshape(shape), loss)


# speedup vs baseline: 1.2904x; 1.1245x over previous
import functools

import jax
import jax.numpy as jnp
from jax import lax
from jax.experimental import pallas as pl
from jax.experimental.pallas import tpu as pltpu
from jax.experimental.pallas import tpu_sc as plsc

_BETA = 0.1


_BM = 512


_KT = 2736


def _round_bf16(x):
    u = lax.bitcast_convert_type(x, jnp.uint32)
    r = u + jnp.uint32(0x7FFF) + ((u >> 16) & jnp.uint32(1))
    return lax.bitcast_convert_type(r & jnp.uint32(0xFFFF0000), jnp.float32)


def _argmin_body(z_ref, emb_ref, z2_ref, e2a_ref, e2b_ref, e2c_ref, idx_ref):
    k = emb_ref.shape[0]
    zb = (z_ref[...] * 2.0).astype(jnp.bfloat16)
    z2 = z2_ref[...]
    acc_v = jnp.full((z_ref.shape[0], 1), jnp.inf, dtype=jnp.float32)
    acc_i = jnp.zeros((z_ref.shape[0], 1), dtype=jnp.int32)
    for t, e2_ref in enumerate((e2a_ref, e2b_ref, e2c_ref)):
        lo = t * _KT
        w = min(_KT, k - lo)
        eb = emb_ref[pl.ds(lo, w), :].astype(jnp.bfloat16)
        dist = lax.dot_general(
            zb, eb,
            dimension_numbers=(((1,), (1,)), ((), ())),
            preferred_element_type=jnp.float32,
        )
        dist = dist + z2
        dist = dist + e2_ref[...]
        i = jnp.argmin(dist, axis=1)[:, None].astype(jnp.int32) + lo
        m = jnp.min(dist, axis=1, keepdims=True)
        keep = (acc_v < m) | ((acc_v == m) & (acc_i < i))
        acc_v = _round_bf16(jnp.where(keep, acc_v, m))
        acc_i = jnp.where(keep, acc_i, i)
    idx_ref[...] = acc_i


def _argmin_call(zf, emb, z2, e2):
    m, c = zf.shape
    k = emb.shape[0]
    grid = (m // _BM,)
    return pl.pallas_call(
        _argmin_body,
        grid=grid,
        in_specs=[
            pl.BlockSpec((_BM, c), lambda i: (i, 0)),
            pl.BlockSpec((k, c), lambda i: (0, 0)),
            pl.BlockSpec((_BM, 1), lambda i: (i, 0)),
            pl.BlockSpec((1, _KT), lambda i: (0, 0)),
            pl.BlockSpec((1, _KT), lambda i: (0, 0)),
            pl.BlockSpec((1, k - 2 * _KT), lambda i: (0, 0)),
        ],
        out_specs=pl.BlockSpec((_BM, 1), lambda i: (i, 0)),
        out_shape=jax.ShapeDtypeStruct((m, 1), jnp.int32),
    )(zf, emb, z2, e2[:, :_KT], e2[:, _KT:2 * _KT], e2[:, 2 * _KT:])


_NC, _NS, _L = 2, 16, 16
_NW = _NC * _NS


def _sc_body(z_hbm, emb_hbm, idx_hbm, out_hbm, part_hbm,
             idx_v, rows_v, z_v, out_v, acc_v,
             gsem0, gsem1, zsem0, zsem1, osem0, osem1):
    m, c = z_hbm.shape
    rows_per_w = m // _NW
    ch = 64
    n_chunks = rows_per_w // ch
    groups = c // _L
    gsems = (gsem0, gsem1)
    zsems = (zsem0, zsem1)
    osems = (osem0, osem1)

    wid = lax.axis_index("s") * _NC + lax.axis_index("c")
    base = wid * rows_per_w
    pltpu.sync_copy(idx_hbm.at[pl.ds(base, rows_per_w)], idx_v)

    def start_in(ci):
        b = ci % 2
        g = pltpu.async_copy(
            emb_hbm.at[idx_v.at[pl.ds(ci * ch, ch)]], rows_v.at[b], gsems[b])
        z = pltpu.async_copy(
            z_hbm.at[pl.ds(base + ci * ch, ch)], z_v.at[b], zsems[b])
        return g, z

    pending = start_in(0)
    out_dmas = [None, None]
    acc = jnp.zeros((_L,), jnp.float32)
    for ci in range(n_chunks):
        b = ci % 2
        g, zc = pending
        if ci + 1 < n_chunks:
            pending = start_in(ci + 1)
        g.wait()
        zc.wait()
        if out_dmas[b] is not None:
            out_dmas[b].wait()

        def do_row(r, acc, b=b):
            for gi in range(groups):
                q = rows_v[b, r, pl.ds(gi * _L, _L)]
                zz = z_v[b, r, pl.ds(gi * _L, _L)]
                d = q - zz
                out_v[b, r, pl.ds(gi * _L, _L)] = zz + d
                acc = acc + d * d
            return acc

        acc = lax.fori_loop(0, ch, do_row, acc)
        out_dmas[b] = pltpu.async_copy(
            out_v.at[b], out_hbm.at[pl.ds(base + ci * ch, ch)], osems[b])
    for dma in out_dmas:
        if dma is not None:
            dma.wait()
    acc_v[...] = acc
    pltpu.sync_copy(acc_v, part_hbm.at[wid])


def _sc_call(zf, emb, idx):
    m, c = zf.shape
    rows_per_w = m // _NW
    ch = 64
    mesh = plsc.VectorSubcoreMesh(core_axis_name="c", subcore_axis_name="s")
    fn = pl.kernel(
        _sc_body,
        out_type=[
            jax.ShapeDtypeStruct((m, c), jnp.float32),
            jax.ShapeDtypeStruct((_NW, _L), jnp.float32),
        ],
        mesh=mesh,
        scratch_types=[
            pltpu.VMEM((rows_per_w,), jnp.int32),
            pltpu.VMEM((2, ch, c), jnp.float32),
            pltpu.VMEM((2, ch, c), jnp.float32),
            pltpu.VMEM((2, ch, c), jnp.float32),
            pltpu.VMEM((_L,), jnp.float32),
            pltpu.SemaphoreType.DMA,
            pltpu.SemaphoreType.DMA,
            pltpu.SemaphoreType.DMA,
            pltpu.SemaphoreType.DMA,
            pltpu.SemaphoreType.DMA,
            pltpu.SemaphoreType.DMA,
        ],
    )
    return fn(zf, emb, idx)


def kernel(z, emb):
    shape = z.shape
    c = shape[-1]
    m = z.size // c
    zf = z.reshape(m, c)
    z2 = jnp.sum(zf ** 2, axis=1, keepdims=True)
    e2 = jnp.sum(emb ** 2, axis=1)[None, :]
    idx = _argmin_call(zf, emb, z2, e2).reshape(m)
    out_f, parts = _sc_call(zf, emb, idx)
    mean_sq = jnp.sum(parts) / (m * c)
    loss = mean_sq + mean_sq * _BETA
    return (out_f.reshape(shape), loss)


# speedup vs baseline: 1.2909x; 1.0004x over previous
import functools

import jax
import jax.numpy as jnp
from jax import lax
from jax.experimental import pallas as pl
from jax.experimental.pallas import tpu as pltpu
from jax.experimental.pallas import tpu_sc as plsc

_BETA = 0.1


_BM = 512


_KT = 2736


def _round_bf16(x):
    u = lax.bitcast_convert_type(x, jnp.uint32)
    r = u + jnp.uint32(0x7FFF) + ((u >> 16) & jnp.uint32(1))
    return lax.bitcast_convert_type(r & jnp.uint32(0xFFFF0000), jnp.float32)


def _argmin_body(z_ref, emb_ref, z2_ref, e2a_ref, e2b_ref, e2c_ref, idx_ref):
    k = emb_ref.shape[0]
    zb = (z_ref[...] * 2.0).astype(jnp.bfloat16)
    z2 = z2_ref[...]
    acc_v = jnp.full((z_ref.shape[0], 1), jnp.inf, dtype=jnp.float32)
    acc_i = jnp.zeros((z_ref.shape[0], 1), dtype=jnp.int32)
    for t, e2_ref in enumerate((e2a_ref, e2b_ref, e2c_ref)):
        lo = t * _KT
        w = min(_KT, k - lo)
        eb = emb_ref[pl.ds(lo, w), :].astype(jnp.bfloat16)
        dist = lax.dot_general(
            zb, eb,
            dimension_numbers=(((1,), (1,)), ((), ())),
            preferred_element_type=jnp.float32,
        )
        dist = dist + z2
        dist = dist + e2_ref[...]
        i = jnp.argmin(dist, axis=1)[:, None].astype(jnp.int32) + lo
        m = jnp.min(dist, axis=1, keepdims=True)
        keep = (acc_v < m) | ((acc_v == m) & (acc_i < i))
        acc_v = _round_bf16(jnp.where(keep, acc_v, m))
        acc_i = jnp.where(keep, acc_i, i)
    idx_ref[...] = acc_i


def _argmin_call(zf, emb, z2, e2):
    m, c = zf.shape
    k = emb.shape[0]
    grid = (m // _BM,)
    return pl.pallas_call(
        _argmin_body,
        grid=grid,
        in_specs=[
            pl.BlockSpec((_BM, c), lambda i: (i, 0)),
            pl.BlockSpec((k, c), lambda i: (0, 0)),
            pl.BlockSpec((_BM, 1), lambda i: (i, 0)),
            pl.BlockSpec((1, _KT), lambda i: (0, 0)),
            pl.BlockSpec((1, _KT), lambda i: (0, 0)),
            pl.BlockSpec((1, k - 2 * _KT), lambda i: (0, 0)),
        ],
        out_specs=pl.BlockSpec((_BM, 1), lambda i: (i, 0)),
        out_shape=jax.ShapeDtypeStruct((m, 1), jnp.int32),
    )(zf, emb, z2, e2[:, :_KT], e2[:, _KT:2 * _KT], e2[:, 2 * _KT:])


_NC, _NS, _L = 2, 16, 16
_NW = _NC * _NS


def _sc_body(z_hbm, emb_hbm, idx_hbm, out_hbm, part_hbm,
             idx_v, rows_v, z_v, out_v, acc_v,
             gsem0, gsem1, zsem0, zsem1, osem0, osem1):
    m, c = z_hbm.shape
    rows_per_w = m // _NW
    ch = 64
    n_chunks = rows_per_w // ch
    groups = c // _L
    gsems = (gsem0, gsem1)
    zsems = (zsem0, zsem1)
    osems = (osem0, osem1)

    wid = lax.axis_index("s") * _NC + lax.axis_index("c")
    base = wid * rows_per_w
    pltpu.sync_copy(idx_hbm.at[pl.ds(base, rows_per_w)], idx_v)

    def start_in(ci):
        b = ci % 2
        g = pltpu.async_copy(
            emb_hbm.at[idx_v.at[pl.ds(ci * ch, ch)]], rows_v.at[b], gsems[b])
        z = pltpu.async_copy(
            z_hbm.at[pl.ds(base + ci * ch, ch)], z_v.at[b], zsems[b])
        return g, z

    pending = start_in(0)
    out_dmas = [None, None]
    accs = tuple(jnp.zeros((_L,), jnp.float32) for _ in range(groups))
    for ci in range(n_chunks):
        b = ci % 2
        g, zc = pending
        if ci + 1 < n_chunks:
            pending = start_in(ci + 1)
        g.wait()
        zc.wait()
        if out_dmas[b] is not None:
            out_dmas[b].wait()

        def do_row(r, accs, b=b):
            new_accs = []
            for gi in range(groups):
                q = rows_v[b, r, pl.ds(gi * _L, _L)]
                zz = z_v[b, r, pl.ds(gi * _L, _L)]
                d = q - zz
                out_v[b, r, pl.ds(gi * _L, _L)] = zz + d
                new_accs.append(accs[gi] + d * d)
            return tuple(new_accs)

        accs = lax.fori_loop(0, ch, do_row, accs)
        out_dmas[b] = pltpu.async_copy(
            out_v.at[b], out_hbm.at[pl.ds(base + ci * ch, ch)], osems[b])
    for dma in out_dmas:
        if dma is not None:
            dma.wait()
    acc = accs[0]
    for a in accs[1:]:
        acc = acc + a
    acc_v[...] = acc
    pltpu.sync_copy(acc_v, part_hbm.at[wid])


def _sc_call(zf, emb, idx):
    m, c = zf.shape
    rows_per_w = m // _NW
    ch = 64
    mesh = plsc.VectorSubcoreMesh(core_axis_name="c", subcore_axis_name="s")
    fn = pl.kernel(
        _sc_body,
        out_type=[
            jax.ShapeDtypeStruct((m, c), jnp.float32),
            jax.ShapeDtypeStruct((_NW, _L), jnp.float32),
        ],
        mesh=mesh,
        scratch_types=[
            pltpu.VMEM((rows_per_w,), jnp.int32),
            pltpu.VMEM((2, ch, c), jnp.float32),
            pltpu.VMEM((2, ch, c), jnp.float32),
            pltpu.VMEM((2, ch, c), jnp.float32),
            pltpu.VMEM((_L,), jnp.float32),
            pltpu.SemaphoreType.DMA,
            pltpu.SemaphoreType.DMA,
            pltpu.SemaphoreType.DMA,
            pltpu.SemaphoreType.DMA,
            pltpu.SemaphoreType.DMA,
            pltpu.SemaphoreType.DMA,
        ],
    )
    return fn(zf, emb, idx)


def kernel(z, emb):
    shape = z.shape
    c = shape[-1]
    m = z.size // c
    zf = z.reshape(m, c)
    z2 = jnp.sum(zf ** 2, axis=1, keepdims=True)
    e2 = jnp.sum(emb ** 2, axis=1)[None, :]
    idx = _argmin_call(zf, emb, z2, e2).reshape(m)
    out_f, parts = _sc_call(zf, emb, idx)
    mean_sq = jnp.sum(parts) / (m * c)
    loss = mean_sq + mean_sq * _BETA
    return (out_f.reshape(shape), loss)
